# Initial kernel scaffold; baseline (speedup 1.0000x reference)
import jax
import jax.numpy as jnp
from jax.experimental import pallas as pl


def kernel(x, gate_w, gate_b, w1, b1, w2, b2):
    raise NotImplementedError("write your pallas kernel here")


# speedup vs baseline: 2.8565x; 2.8565x over previous
import functools

import jax
import jax.numpy as jnp
from jax.experimental import pallas as pl
from jax.experimental.pallas import tpu as pltpu

D = 1024
H = 4096
NE = 8
K = 2
TILE = 256
GB = 512


def _gate_body(x_ref, gw_ref, gb_ref, wi_ref, ww_ref):
    logits = jnp.dot(x_ref[...], gw_ref[...], preferred_element_type=jnp.float32)
    logits = logits + gb_ref[...]
    col = jax.lax.broadcasted_iota(jnp.int32, logits.shape, 1)
    neg = jnp.float32(-1e30)
    lm = jnp.where(col < NE, logits, neg)
    m1 = jnp.max(lm, axis=1, keepdims=True)
    i1 = jnp.min(jnp.where(lm == m1, col, 10**9), axis=1, keepdims=True)
    lm2 = jnp.where(col == i1, neg, lm)
    m2 = jnp.max(lm2, axis=1, keepdims=True)
    i2 = jnp.min(jnp.where(lm2 == m2, col, 10**9), axis=1, keepdims=True)
    w1g = 1.0 / (1.0 + jnp.exp(m2 - m1))
    wi_ref[...] = jnp.where(col == 0, i1, jnp.where(col == 1, i2, 0)).astype(jnp.int32)
    ww_ref[...] = jnp.where(col == 0, w1g, jnp.where(col == 1, 1.0 - w1g, 0.0))


def _moe_body(te_ref, tm_ref, nv_ref, xs_ref, w1_ref, b1_ref, w2_ref, b2_ref,
              pw_ref, out_ref):
    t = pl.program_id(0)

    @pl.when(t < nv_ref[0])
    def _():
        xt = xs_ref[...].astype(jnp.bfloat16)
        h = jnp.dot(xt, w1_ref[0], preferred_element_type=jnp.float32)
        h = h + b1_ref[0]
        h = 0.5 * h * (1.0 + jax.lax.erf(h * 0.7071067811865476))
        o = jnp.dot(h.astype(jnp.bfloat16), w2_ref[0],
                    preferred_element_type=jnp.float32)
        o = (o + b2_ref[0]) * pw_ref[...]
        out_ref[...] = o


def _gate(xf, gate_w, gate_b):
    n = xf.shape[0]
    gwp = jnp.pad(gate_w, ((0, 0), (0, 128 - NE)))
    gbp = jnp.pad(gate_b, (0, 128 - NE)).reshape(1, 128)
    ti, tw = pl.pallas_call(
        _gate_body,
        grid=(n // GB,),
        in_specs=[
            pl.BlockSpec((GB, D), lambda i: (i, 0)),
            pl.BlockSpec((D, 128), lambda i: (0, 0)),
            pl.BlockSpec((1, 128), lambda i: (0, 0)),
        ],
        out_specs=[
            pl.BlockSpec((GB, 128), lambda i: (i, 0)),
            pl.BlockSpec((GB, 128), lambda i: (i, 0)),
        ],
        out_shape=[
            jax.ShapeDtypeStruct((n, 128), jnp.int32),
            jax.ShapeDtypeStruct((n, 128), jnp.float32),
        ],
    )(xf, gwp, gbp)
    return ti[:, :K], tw[:, :K]


def kernel(x, gate_w, gate_b, w1, b1, w2, b2):
    bsh = x.shape
    n = bsh[0] * bsh[1]
    nk = n * K
    p = nk + NE * TILE
    nt = p // TILE
    xf = x.reshape(n, D)

    top_i, top_w = _gate(xf, gate_w, gate_b)

    eid = top_i.reshape(nk)
    order = jnp.argsort(eid, stable=True)
    es = eid[order]
    counts = jnp.bincount(eid, length=NE)
    padc = ((counts + TILE - 1) // TILE) * TILE
    cpad = jnp.cumsum(padc)
    off = cpad - padc
    ccum = jnp.cumsum(counts) - counts
    rank = jnp.arange(nk, dtype=jnp.int32) - ccum[es].astype(jnp.int32)
    pos = off[es].astype(jnp.int32) + rank
    ptok = jnp.zeros(p, jnp.int32).at[pos].set((order // K).astype(jnp.int32))
    pwt = jnp.zeros(p, jnp.float32).at[pos].set(top_w.reshape(nk)[order])
    dest = jnp.zeros(nk, jnp.int32).at[order].set(pos)
    nv = (cpad[-1] // TILE).astype(jnp.int32)
    tidx = jnp.arange(nt, dtype=jnp.int32)
    te = jnp.searchsorted(cpad, tidx * TILE, side="right").astype(jnp.int32)
    te = jnp.where(tidx < nv, jnp.clip(te, 0, NE - 1), te[nv - 1])
    tm = jnp.minimum(tidx, nv - 1)

    xs = xf[ptok]
    pwcol = pwt[:, None]

    grid_spec = pltpu.PrefetchScalarGridSpec(
        num_scalar_prefetch=3,
        grid=(nt,),
        in_specs=[
            pl.BlockSpec((TILE, D), lambda t, te_, tm_, nv_: (tm_[t], 0)),
            pl.BlockSpec((1, D, H), lambda t, te_, tm_, nv_: (te_[t], 0, 0)),
            pl.BlockSpec((1, 1, H), lambda t, te_, tm_, nv_: (te_[t], 0, 0)),
            pl.BlockSpec((1, H, D), lambda t, te_, tm_, nv_: (te_[t], 0, 0)),
            pl.BlockSpec((1, 1, D), lambda t, te_, tm_, nv_: (te_[t], 0, 0)),
            pl.BlockSpec((TILE, 1), lambda t, te_, tm_, nv_: (tm_[t], 0)),
        ],
        out_specs=pl.BlockSpec((TILE, D), lambda t, te_, tm_, nv_: (tm_[t], 0)),
    )
    pair_out = pl.pallas_call(
        _moe_body,
        grid_spec=grid_spec,
        out_shape=jax.ShapeDtypeStruct((p, D), jnp.float32),
    )(te, tm, nv.reshape(1), xs,
      w1.astype(jnp.bfloat16), b1.reshape(NE, 1, H),
      w2.astype(jnp.bfloat16), b2.reshape(NE, 1, D), pwcol)

    d2 = dest.reshape(n, K)
    final = pair_out[d2[:, 0]] + pair_out[d2[:, 1]]
    return final.reshape(bsh)
